# baseline (speedup 1.0000x reference)
import dataclasses
import functools

import jax
import jax.numpy as jnp
from jax import lax
from jax.experimental import pallas as pl
from jax.experimental.pallas import tpu as pltpu
from jax.experimental.pallas import tpu_sc as plsc

N = 10000
NPAD = 10240
E = 160000
CHUNK = 128
EPAD = 163840
NCHUNKS = EPAD // CHUNK
NTILES = 16
ROWS_PER_TILE = NPAD // NTILES
ZROWS = 128
NBUF = 2
F32 = jnp.float32


def _segsum_sc(p, src2, dst2, ew):
    Dc = 128
    n_j = Dc // 16
    nch_t = NCHUNKS // (2 * NTILES)
    mesh = plsc.VectorSubcoreMesh(core_axis_name="c", subcore_axis_name="s")

    def body(*refs):
        (p_hbm, src_hbm, dst_hbm, ew_hbm, o0_hbm, o1_hbm,
         src_v, dst_v, ew_v, acc, *bufsems) = refs
        bufs = bufsems[:NBUF]
        gsems = bufsems[NBUF:2 * NBUF]
        ssems = bufsems[2 * NBUF:3 * NBUF]
        c = lax.axis_index("c")
        s = lax.axis_index("s")

        @pl.loop(0, ZROWS)
        def _(i):
            for j in range(n_j):
                bufs[0][i, pl.ds(j * 16, 16)] = jnp.zeros((16,), F32)
        for k in range(ROWS_PER_TILE // ZROWS):
            pltpu.sync_copy(
                bufs[0], acc.at[pl.ds(s * ROWS_PER_TILE + k * ZROWS, ZROWS)])

        tstart = (c * NTILES + s) * nch_t
        pltpu.sync_copy(src_hbm.at[pl.ds(tstart, nch_t)], src_v)
        pltpu.sync_copy(dst_hbm.at[pl.ds(tstart, nch_t)], dst_v)
        pltpu.sync_copy(ew_hbm.at[pl.ds(tstart * CHUNK, nch_t * CHUNK)], ew_v)
        plsc.subcore_barrier()

        def issue_gather(t, b):
            pltpu.async_copy(p_hbm.at[src_v.at[t]], bufs[b], gsems[b])

        def wait_gather(b):
            pltpu.make_async_copy(
                p_hbm.at[src_v.at[0]], bufs[b], gsems[b]).wait()

        def issue_scatter(t, b):
            pltpu.async_copy(bufs[b], acc.at[dst_v.at[t]], ssems[b],
                             add=True)

        def wait_scatter(b):
            pltpu.make_async_copy(
                bufs[b], acc.at[dst_v.at[0]], ssems[b]).wait()

        for b in range(NBUF):
            issue_gather(b, b)

        @pl.loop(0, nch_t, step=NBUF)
        def _(g):
            for b in range(NBUF):
                t = g + b
                wait_gather(b)

                nb = (b + NBUF - 1) % NBUF
                ok = jnp.logical_and(t >= 1, t + NBUF - 1 < nch_t)
                def retire_prefetch(t=t, nb=nb):
                    wait_scatter(nb)
                    issue_gather(t + NBUF - 1, nb)
                pl.when(ok)(retire_prefetch)

                @plsc.parallel_loop(0, CHUNK, unroll=4)
                def _(e):
                    w = plsc.load_gather(
                        ew_v, [jnp.full((16,), t * CHUNK + e, jnp.int32)])
                    for j in range(n_j):
                        sl = (e, pl.ds(j * 16, 16))
                        bufs[b][sl] = bufs[b][sl] * w

                issue_scatter(t, b)

        for b in range(NBUF):
            wait_scatter(b)

        plsc.subcore_barrier()
        rsl = pl.ds(s * ROWS_PER_TILE, ROWS_PER_TILE)
        pl.when(c == 0)(lambda: pltpu.sync_copy(acc.at[rsl], o0_hbm.at[rsl]))
        pl.when(c == 1)(lambda: pltpu.sync_copy(acc.at[rsl], o1_hbm.at[rsl]))

    out_t = (jax.ShapeDtypeStruct((NPAD, Dc), F32),
             jax.ShapeDtypeStruct((NPAD, Dc), F32))
    cp = pltpu.CompilerParams()
    if "needs_layout_passes" in pltpu.CompilerParams.__dataclass_fields__:
        cp = dataclasses.replace(cp, needs_layout_passes=False)
    fn = pl.kernel(
        body,
        out_type=out_t,
        mesh=mesh,
        compiler_params=cp,
        scratch_types=(
            [pltpu.VMEM((nch_t, CHUNK), jnp.int32),
             pltpu.VMEM((nch_t, CHUNK), jnp.int32),
             pltpu.VMEM((nch_t * CHUNK,), F32),
             pltpu.VMEM_SHARED((NPAD, Dc), F32)]
            + [pltpu.VMEM((CHUNK, Dc), F32)] * NBUF
            + [pltpu.SemaphoreType.DMA] * (2 * NBUF)
        ),
    )
    return fn(p, src2, dst2, ew)


BN = 2000
GRID = N // BN


def _bs(shape, im):
    return pl.BlockSpec(shape, im)


def _row(i):
    return (i, 0)


def _rep(i):
    return (0, 0)


def _tc_project1(x, Wr):
    D = Wr.shape[1]
    Dh = D // 2
    K = x.shape[1]

    def body(x_ref, wr_ref, p0_ref, p1_ref):
        p = jnp.dot(x_ref[...], wr_ref[...], preferred_element_type=F32)
        p0_ref[...] = p[:, :Dh]
        p1_ref[...] = p[:, Dh:]

    return pl.pallas_call(
        body,
        grid=(GRID,),
        in_specs=[_bs((BN, K), _row), _bs((K, D), _rep)],
        out_specs=[_bs((BN, Dh), _row), _bs((BN, Dh), _row)],
        out_shape=[jax.ShapeDtypeStruct((N, Dh), F32),
                   jax.ShapeDtypeStruct((N, Dh), F32)],
    )(x, Wr)


def _tc_root1(x, Ws, b):
    D = Ws.shape[1]
    K = x.shape[1]

    def body(x_ref, ws_ref, b_ref, r_ref):
        r_ref[...] = jnp.dot(x_ref[...], ws_ref[...],
                             preferred_element_type=F32) + b_ref[...]

    return pl.pallas_call(
        body,
        grid=(GRID,),
        in_specs=[_bs((BN, K), _row), _bs((K, D), _rep), _bs((1, D), _rep)],
        out_specs=_bs((BN, D), _row),
        out_shape=jax.ShapeDtypeStruct((N, D), F32),
    )(x, Ws, b.reshape(1, D))


def _tc_mid(a0p, a1p, r, Wr):
    Dh_in = a0p[0].shape[1]
    D = Wr.shape[1]

    def body(a00, a01, a10, a11, r_ref, wr_ref, p_ref, h_ref):
        h = jnp.concatenate([a00[...] + a01[...], a10[...] + a11[...]],
                            axis=1) + r_ref[...]
        h = jnp.maximum(h, 0.0)
        h_ref[...] = h
        p_ref[...] = jnp.dot(h, wr_ref[...], preferred_element_type=F32)

    K = 2 * Dh_in
    return pl.pallas_call(
        body,
        grid=(GRID,),
        in_specs=[_bs((BN, Dh_in), _row)] * 4 +
                 [_bs((BN, K), _row), _bs((K, D), _rep)],
        out_specs=[_bs((BN, D), _row), _bs((BN, K), _row)],
        out_shape=[jax.ShapeDtypeStruct((N, D), F32),
                   jax.ShapeDtypeStruct((N, K), F32)],
    )(*a0p, *a1p, r, Wr)


def _tc_last_h(a0, a1, r):
    D = a0.shape[1]

    def body(a0_ref, a1_ref, r_ref, h_ref):
        h_ref[...] = jnp.maximum(a0_ref[...] + a1_ref[...] + r_ref[...], 0.0)

    return pl.pallas_call(
        body,
        grid=(GRID,),
        in_specs=[_bs((BN, D), _row), _bs((BN, D), _row), _bs((BN, D), _row)],
        out_specs=_bs((BN, D), _row),
        out_shape=jax.ShapeDtypeStruct((N, D), F32),
    )(a0, a1, r)


def _tc_final(b0, b1, r3, Wr):
    D = b0.shape[1]
    Do = Wr.shape[1]

    def body(b0_ref, b1_ref, r_ref, wr_ref, o_ref):
        agg = b0_ref[...] + b1_ref[...]
        o_ref[...] = jnp.dot(agg, wr_ref[...], preferred_element_type=F32) + r_ref[...]

    return pl.pallas_call(
        body,
        grid=(GRID,),
        in_specs=[_bs((BN, D), _row), _bs((BN, D), _row), _bs((BN, Do), _row),
                  _bs((D, Do), _rep)],
        out_specs=_bs((BN, Do), _row),
        out_shape=jax.ShapeDtypeStruct((N, Do), F32),
    )(b0, b1, r3, Wr)


def kernel(x, edge_index, edge_weight, batch, Wr1, br1, Ws1,
           Wr2, br2, Ws2, Wr3, br3, Ws3):
    del batch
    pad = EPAD - E
    pad_src = (jnp.arange(pad, dtype=jnp.int32) * 61) % N
    pad_dst = N + (jnp.arange(pad, dtype=jnp.int32) % (NPAD - N))
    src2 = jnp.concatenate([edge_index[0], pad_src]).reshape(NCHUNKS, CHUNK)
    dst2 = jnp.concatenate([edge_index[1], pad_dst]).reshape(NCHUNKS, CHUNK)
    ewp = jnp.pad(edge_weight, (0, pad))

    p0, p1 = _tc_project1(x, Wr1)
    a0p = _segsum_sc(p0, src2, dst2, ewp)
    a1p = _segsum_sc(p1, src2, dst2, ewp)
    r1 = _tc_root1(x, Ws1, br1)

    p2, h1 = _tc_mid(a0p, a1p, r1, Wr2)
    a0, a1 = _segsum_sc(p2, src2, dst2, ewp)
    r2 = _tc_root1(h1, Ws2, br2)

    h2 = _tc_last_h(a0, a1, r2)
    b0, b1 = _segsum_sc(h2, src2, dst2, ewp)
    r3 = _tc_root1(h2, Ws3, br3)

    return _tc_final(b0, b1, r3, Wr3)


# speedup vs baseline: 1.0038x; 1.0038x over previous
import dataclasses

import jax
import jax.numpy as jnp
from jax import lax
from jax.experimental import pallas as pl
from jax.experimental.pallas import tpu as pltpu
from jax.experimental.pallas import tpu_sc as plsc

N = 10000
NPAD = 10240
E = 160000
CHUNK = 128
EPAD = 163840
NCHUNKS = EPAD // CHUNK
NTILES = 16
ROWS_PER_TILE = NPAD // NTILES
ZROWS = 128
NBUF = 2
F32 = jnp.float32


def _segsum_sc(p, src2, dst2, ew):
    Dc = 128
    n_j = Dc // 16
    nch_t = NCHUNKS // (2 * NTILES)
    mesh = plsc.VectorSubcoreMesh(core_axis_name="c", subcore_axis_name="s")

    def body(*refs):
        (p_hbm, src_hbm, dst_hbm, ew_hbm, o0_hbm, o1_hbm,
         src_v, dst_v, ew_v, acc, *bufsems) = refs
        bufs = bufsems[:NBUF]
        gsems = bufsems[NBUF:2 * NBUF]
        ssems = bufsems[2 * NBUF:3 * NBUF]
        c = lax.axis_index("c")
        s = lax.axis_index("s")

        @pl.loop(0, ZROWS)
        def _(i):
            for j in range(n_j):
                bufs[0][i, pl.ds(j * 16, 16)] = jnp.zeros((16,), F32)
        for k in range(ROWS_PER_TILE // ZROWS):
            pltpu.sync_copy(
                bufs[0], acc.at[pl.ds(s * ROWS_PER_TILE + k * ZROWS, ZROWS)])

        tstart = (c * NTILES + s) * nch_t
        pltpu.sync_copy(src_hbm.at[pl.ds(tstart, nch_t)], src_v)
        pltpu.sync_copy(dst_hbm.at[pl.ds(tstart, nch_t)], dst_v)
        pltpu.sync_copy(ew_hbm.at[pl.ds(tstart * CHUNK, nch_t * CHUNK)], ew_v)
        plsc.subcore_barrier()

        def issue_gather(t, b):
            pltpu.async_copy(p_hbm.at[src_v.at[t]], bufs[b], gsems[b])

        def wait_gather(b):
            pltpu.make_async_copy(
                p_hbm.at[src_v.at[0]], bufs[b], gsems[b]).wait()

        def issue_scatter(t, b):
            pltpu.async_copy(bufs[b], acc.at[dst_v.at[t]], ssems[b],
                             add=True)

        def wait_scatter(b):
            pltpu.make_async_copy(
                bufs[b], acc.at[dst_v.at[0]], ssems[b]).wait()

        for b in range(NBUF):
            issue_gather(b, b)

        @pl.loop(0, nch_t, step=NBUF)
        def _(g):
            for b in range(NBUF):
                t = g + b
                wait_gather(b)

                nb = (b + NBUF - 1) % NBUF
                ok = jnp.logical_and(t >= 1, t + NBUF - 1 < nch_t)
                def retire_prefetch(t=t, nb=nb):
                    wait_scatter(nb)
                    issue_gather(t + NBUF - 1, nb)
                pl.when(ok)(retire_prefetch)

                @plsc.parallel_loop(0, CHUNK, unroll=4)
                def _(e):
                    w = plsc.load_gather(
                        ew_v, [jnp.full((16,), t * CHUNK + e, jnp.int32)])
                    for j in range(n_j):
                        sl = (e, pl.ds(j * 16, 16))
                        bufs[b][sl] = bufs[b][sl] * w

                issue_scatter(t, b)

        for b in range(NBUF):
            wait_scatter(b)

        plsc.subcore_barrier()
        rsl = pl.ds(s * ROWS_PER_TILE, ROWS_PER_TILE)
        pl.when(c == 0)(lambda: pltpu.sync_copy(acc.at[rsl], o0_hbm.at[rsl]))
        pl.when(c == 1)(lambda: pltpu.sync_copy(acc.at[rsl], o1_hbm.at[rsl]))

    out_t = (jax.ShapeDtypeStruct((NPAD, Dc), F32),
             jax.ShapeDtypeStruct((NPAD, Dc), F32))
    cp = pltpu.CompilerParams()
    if "needs_layout_passes" in pltpu.CompilerParams.__dataclass_fields__:
        cp = dataclasses.replace(cp, needs_layout_passes=False)
    fn = pl.kernel(
        body,
        out_type=out_t,
        mesh=mesh,
        compiler_params=cp,
        scratch_types=(
            [pltpu.VMEM((nch_t, CHUNK), jnp.int32),
             pltpu.VMEM((nch_t, CHUNK), jnp.int32),
             pltpu.VMEM((nch_t * CHUNK,), F32),
             pltpu.VMEM_SHARED((NPAD, Dc), F32)]
            + [pltpu.VMEM((CHUNK, Dc), F32)] * NBUF
            + [pltpu.SemaphoreType.DMA] * (2 * NBUF)
        ),
    )
    return fn(p, src2, dst2, ew)


BN = 2000
GRID = N // BN


def _bs(shape, im):
    return pl.BlockSpec(shape, im)


def _row(i):
    return (i, 0)


def _rep(i):
    return (0, 0)


def _tc_project1(x, Wr):
    D = Wr.shape[1]
    Dh = D // 2
    K = x.shape[1]

    def body(x_ref, wr_ref, p0_ref, p1_ref):
        p = jnp.dot(x_ref[...], wr_ref[...], preferred_element_type=F32)
        p0_ref[...] = p[:, :Dh]
        p1_ref[...] = p[:, Dh:]

    return pl.pallas_call(
        body,
        grid=(GRID,),
        in_specs=[_bs((BN, K), _row), _bs((K, D), _rep)],
        out_specs=[_bs((BN, Dh), _row), _bs((BN, Dh), _row)],
        out_shape=[jax.ShapeDtypeStruct((N, Dh), F32),
                   jax.ShapeDtypeStruct((N, Dh), F32)],
    )(x, Wr)


def _tc_root1(x, Ws, b):
    D = Ws.shape[1]
    K = x.shape[1]

    def body(x_ref, ws_ref, b_ref, r_ref):
        r_ref[...] = jnp.dot(x_ref[...], ws_ref[...],
                             preferred_element_type=F32) + b_ref[...]

    return pl.pallas_call(
        body,
        grid=(GRID,),
        in_specs=[_bs((BN, K), _row), _bs((K, D), _rep), _bs((1, D), _rep)],
        out_specs=_bs((BN, D), _row),
        out_shape=jax.ShapeDtypeStruct((N, D), F32),
    )(x, Ws, b.reshape(1, D))


def _tc_mid(a0p, a1p, r, Wr):
    Dh_in = a0p[0].shape[1]
    D = Wr.shape[1]

    def body(a00, a01, a10, a11, r_ref, wr_ref, p_ref, h_ref):
        h = jnp.concatenate([a00[...] + a01[...], a10[...] + a11[...]],
                            axis=1) + r_ref[...]
        h = jnp.maximum(h, 0.0)
        h_ref[...] = h
        p_ref[...] = jnp.dot(h, wr_ref[...], preferred_element_type=F32)

    K = 2 * Dh_in
    return pl.pallas_call(
        body,
        grid=(GRID,),
        in_specs=[_bs((BN, Dh_in), _row)] * 4 +
                 [_bs((BN, K), _row), _bs((K, D), _rep)],
        out_specs=[_bs((BN, D), _row), _bs((BN, K), _row)],
        out_shape=[jax.ShapeDtypeStruct((N, D), F32),
                   jax.ShapeDtypeStruct((N, K), F32)],
    )(*a0p, *a1p, r, Wr)


def _tc_last_h(a0, a1, r):
    D = a0.shape[1]

    def body(a0_ref, a1_ref, r_ref, h_ref):
        h_ref[...] = jnp.maximum(a0_ref[...] + a1_ref[...] + r_ref[...], 0.0)

    return pl.pallas_call(
        body,
        grid=(GRID,),
        in_specs=[_bs((BN, D), _row), _bs((BN, D), _row), _bs((BN, D), _row)],
        out_specs=_bs((BN, D), _row),
        out_shape=jax.ShapeDtypeStruct((N, D), F32),
    )(a0, a1, r)


def _tc_final(b0, b1, r3, Wr):
    D = b0.shape[1]
    Do = Wr.shape[1]

    def body(b0_ref, b1_ref, r_ref, wr_ref, o_ref):
        agg = b0_ref[...] + b1_ref[...]
        o_ref[...] = jnp.dot(agg, wr_ref[...], preferred_element_type=F32) + r_ref[...]

    return pl.pallas_call(
        body,
        grid=(GRID,),
        in_specs=[_bs((BN, D), _row), _bs((BN, D), _row), _bs((BN, Do), _row),
                  _bs((D, Do), _rep)],
        out_specs=_bs((BN, Do), _row),
        out_shape=jax.ShapeDtypeStruct((N, Do), F32),
    )(b0, b1, r3, Wr)


def kernel(x, edge_index, edge_weight, batch, Wr1, br1, Ws1,
           Wr2, br2, Ws2, Wr3, br3, Ws3):
    del batch
    pad = EPAD - E
    pad_src = (jnp.arange(pad, dtype=jnp.int32) * 61) % N
    pad_dst = N + (jnp.arange(pad, dtype=jnp.int32) % (NPAD - N))
    src2 = jnp.concatenate([edge_index[0], pad_src]).reshape(NCHUNKS, CHUNK)
    dst2 = jnp.concatenate([edge_index[1], pad_dst]).reshape(NCHUNKS, CHUNK)
    ewp = jnp.pad(edge_weight, (0, pad))

    p0, p1 = _tc_project1(x, Wr1)
    a0p = _segsum_sc(p0, src2, dst2, ewp)
    a1p = _segsum_sc(p1, src2, dst2, ewp)
    r1 = _tc_root1(x, Ws1, br1)

    p2, h1 = _tc_mid(a0p, a1p, r1, Wr2)
    a0, a1 = _segsum_sc(p2, src2, dst2, ewp)
    r2 = _tc_root1(h1, Ws2, br2)

    h2 = _tc_last_h(a0, a1, r2)
    b0, b1 = _segsum_sc(h2, src2, dst2, ewp)
    r3 = _tc_root1(h2, Ws3, br3)

    return _tc_final(b0, b1, r3, Wr3)
